# baseline (speedup 1.0000x reference)
import functools
import jax
import jax.numpy as jnp
from jax import lax
from jax.experimental import pallas as pl
from jax.experimental.pallas import tpu as pltpu
from jax.experimental.pallas import tpu_sc as plsc

D_EMBED = 2048
N_EXP = 16
CAP = 512
N_BATCH = 2
S_SEQ = 4096

ROW_TILE = 512
N_TILES_PER_B = S_SEQ // ROW_TILE

LANES = 16
N_CHUNK = S_SEQ // LANES
UNROLL = 16
COLS_PER_SUB = S_SEQ // 16


def _router_body(x_ref, wt_ref, logits_ref, probs_ref, probsT_ref):
    l = jnp.dot(x_ref[...], wt_ref[...], preferred_element_type=jnp.float32)
    m = jnp.max(l, axis=-1, keepdims=True)
    e = jnp.exp(l - m)
    p = e / jnp.sum(e, axis=-1, keepdims=True)
    logits_ref[...] = l
    probs_ref[...] = p
    probsT_ref[...] = jax.lax.bitcast_convert_type(p.T, jnp.int32)


def _sc_mask_body(probsT_hbm, out_hbm, col_v, sel_v, stage_v, acc_buf, un_v):
    b = lax.axis_index("c")
    e = lax.axis_index("s")

    pltpu.sync_copy(probsT_hbm.at[b, e], col_v)

    zero_v = jnp.zeros((LANES,), jnp.int32)
    lane_iota = lax.iota(jnp.int32, LANES)

    def _lane_total(acc):
        tot = acc[0]
        for j in range(1, LANES):
            tot = tot + acc[j]
        return tot

    def count_ge(thr_s):
        thr_v = jnp.full((LANES,), thr_s, jnp.int32)

        def body(i, acc):
            base = i * (LANES * UNROLL)
            for u in range(UNROLL):
                v = col_v[pl.ds(base + u * LANES, LANES)]
                acc = acc + jnp.where(v >= thr_v, 1, 0).astype(jnp.int32)
            return acc

        return _lane_total(
            lax.fori_loop(0, N_CHUNK // UNROLL, body, zero_v)
        )

    def bstep(_, state):
        lo, hi, c_lo, c_hi = state
        mid = lo + lax.shift_right_arithmetic(hi - lo, 1)
        c = count_ge(mid)
        ge = c >= CAP
        return (
            jnp.where(ge, mid, lo),
            jnp.where(ge, hi, mid),
            jnp.where(ge, c, c_lo),
            jnp.where(ge, c_hi, c),
        )

    lo, _, c_lo, c_hi = lax.fori_loop(
        0,
        31,
        bstep,
        (jnp.int32(0), jnp.int32(0x3F800001), jnp.int32(S_SEQ), jnp.int32(0)),
    )
    thr_v = jnp.full((LANES,), lo, jnp.int32)

    n_gt = c_hi
    n_eq = c_lo - c_hi
    rem = CAP - n_gt

    def count_eq_le(icut_s):
        icut_v = jnp.full((LANES,), icut_s, jnp.int32)

        def body(i, acc):
            base = i * (LANES * UNROLL)
            for u in range(UNROLL):
                off = base + u * LANES
                v = col_v[pl.ds(off, LANES)]
                m = (v == thr_v) & ((lane_iota + off) <= icut_v)
                acc = acc + jnp.where(m, 1, 0).astype(jnp.int32)
            return acc

        return _lane_total(
            lax.fori_loop(0, N_CHUNK // UNROLL, body, zero_v)
        )

    def istep(_, lohi):
        lo, hi = lohi
        mid = lo + lax.shift_right_arithmetic(hi - lo + 1, 1)
        ok = count_eq_le(mid) >= rem
        return (jnp.where(ok, lo, mid), jnp.where(ok, mid, hi))

    def _tie_search():
        _, icut = lax.fori_loop(
            0, 13, istep, (jnp.int32(-1), jnp.int32(S_SEQ - 1))
        )
        return icut

    icut = lax.cond(n_eq == rem, lambda: jnp.int32(S_SEQ - 1), _tie_search)
    icut_v = jnp.full((LANES,), icut, jnp.int32)

    one_f = jnp.ones((LANES,), jnp.float32)
    zero_f = jnp.zeros((LANES,), jnp.float32)

    def selbody(i, carry):
        base = i * (LANES * UNROLL)
        for u in range(UNROLL):
            off = base + u * LANES
            v = col_v[pl.ds(off, LANES)]
            take = (v > thr_v) | ((v == thr_v) & ((lane_iota + off) <= icut_v))
            sel_v[pl.ds(off, LANES)] = jnp.where(take, one_f, zero_f)
        return carry

    lax.fori_loop(0, N_CHUNK // UNROLL, selbody, jnp.int32(0))

    pltpu.sync_copy(sel_v, un_v.at[e])
    plsc.subcore_barrier()

    base = e * COLS_PER_SUB
    pltpu.sync_copy(un_v.at[:, pl.ds(base, COLS_PER_SUB)], stage_v)
    for k in range(COLS_PER_SUB // LANES):
        acc = stage_v[0, pl.ds(k * LANES, LANES)]
        for r in range(1, 16):
            acc = jnp.maximum(acc, stage_v[r, pl.ds(k * LANES, LANES)])
        sel_v[pl.ds(k * LANES, LANES)] = acc
    pltpu.sync_copy(sel_v.at[pl.ds(0, COLS_PER_SUB)], out_hbm.at[b, pl.ds(base, COLS_PER_SUB)])


_sc_mask = functools.partial(
    pl.kernel,
    out_type=jax.ShapeDtypeStruct((N_BATCH, S_SEQ), jnp.float32),
    mesh=plsc.VectorSubcoreMesh(core_axis_name="c", subcore_axis_name="s"),
    scratch_types=[
        pltpu.VMEM((S_SEQ,), jnp.int32),
        pltpu.VMEM((S_SEQ,), jnp.float32),
        pltpu.VMEM((16, COLS_PER_SUB), jnp.float32),
        pltpu.VMEM((LANES,), jnp.int32),
        pltpu.VMEM_SHARED((16, S_SEQ), jnp.float32),
    ],
)(_sc_mask_body)


@jax.jit
def kernel(x, W):
    xr = x.reshape(N_BATCH * S_SEQ, D_EMBED)
    wt = W.T

    n_tiles = (N_BATCH * S_SEQ) // ROW_TILE
    logits_r, probs_r, probsT_r = pl.pallas_call(
        _router_body,
        grid=(n_tiles,),
        in_specs=[
            pl.BlockSpec((ROW_TILE, D_EMBED), lambda i: (i, 0)),
            pl.BlockSpec((D_EMBED, N_EXP), lambda i: (0, 0)),
        ],
        out_specs=[
            pl.BlockSpec((ROW_TILE, N_EXP), lambda i: (i, 0)),
            pl.BlockSpec((ROW_TILE, N_EXP), lambda i: (i, 0)),
            pl.BlockSpec(
                (N_EXP, ROW_TILE),
                lambda i: (i // N_TILES_PER_B, i % N_TILES_PER_B),
            ),
        ],
        out_shape=[
            jax.ShapeDtypeStruct((N_BATCH * S_SEQ, N_EXP), jnp.float32),
            jax.ShapeDtypeStruct((N_BATCH * S_SEQ, N_EXP), jnp.float32),
            jax.ShapeDtypeStruct((N_BATCH * N_EXP, S_SEQ), jnp.int32),
        ],
    )(xr, wt)

    logits = logits_r.reshape(N_BATCH, S_SEQ, N_EXP)
    probs = probs_r.reshape(N_BATCH, S_SEQ, N_EXP)
    probsT = probsT_r.reshape(N_BATCH, N_EXP, S_SEQ)

    sel = _sc_mask(probsT)
    mask = jnp.concatenate(
        [sel[:, :, None], jnp.zeros((N_BATCH, S_SEQ, N_EXP - 1), jnp.float32)],
        axis=-1,
    )

    return (mask, probs, logits)


# speedup vs baseline: 1.0026x; 1.0026x over previous
import functools
import jax
import jax.numpy as jnp
from jax import lax
from jax.experimental import pallas as pl
from jax.experimental.pallas import tpu as pltpu
from jax.experimental.pallas import tpu_sc as plsc

D_EMBED = 2048
N_EXP = 16
CAP = 512
N_BATCH = 2
S_SEQ = 4096

ROW_TILE = 512
N_TILES_PER_B = S_SEQ // ROW_TILE

LANES = 16
N_CHUNK = S_SEQ // LANES
UNROLL = 8
COLS_PER_SUB = S_SEQ // 16


def _router_body(x_ref, wt_ref, logits_ref, probs_ref, probsT_ref):
    l = jnp.dot(x_ref[...], wt_ref[...], preferred_element_type=jnp.float32)
    m = jnp.max(l, axis=-1, keepdims=True)
    e = jnp.exp(l - m)
    p = e / jnp.sum(e, axis=-1, keepdims=True)
    logits_ref[...] = l
    probs_ref[...] = p
    probsT_ref[...] = jax.lax.bitcast_convert_type(p.T, jnp.int32)


def _sc_mask_body(probsT_hbm, out_hbm, col_v, sel_v, stage_v, acc_buf, un_v):
    b = lax.axis_index("c")
    e = lax.axis_index("s")

    pltpu.sync_copy(probsT_hbm.at[b, e], col_v)

    zero_v = jnp.zeros((LANES,), jnp.int32)
    lane_iota = lax.iota(jnp.int32, LANES)

    def _lane_total(acc):
        tot = acc[0]
        for j in range(1, LANES):
            tot = tot + acc[j]
        return tot

    def count_ge(thr_s):
        thr_v = jnp.full((LANES,), thr_s, jnp.int32)

        def body(i, acc):
            base = i * (LANES * UNROLL)
            for u in range(UNROLL):
                v = col_v[pl.ds(base + u * LANES, LANES)]
                acc = acc + jnp.where(v >= thr_v, 1, 0).astype(jnp.int32)
            return acc

        return _lane_total(
            lax.fori_loop(0, N_CHUNK // UNROLL, body, zero_v)
        )

    def bstep(_, state):
        lo, hi, c_lo, c_hi = state
        mid = lo + lax.shift_right_arithmetic(hi - lo, 1)
        c = count_ge(mid)
        ge = c >= CAP
        return (
            jnp.where(ge, mid, lo),
            jnp.where(ge, hi, mid),
            jnp.where(ge, c, c_lo),
            jnp.where(ge, c_hi, c),
        )

    lo, _, c_lo, c_hi = lax.fori_loop(
        0,
        30,
        bstep,
        (jnp.int32(0), jnp.int32(0x3F800001), jnp.int32(S_SEQ), jnp.int32(0)),
    )
    thr_v = jnp.full((LANES,), lo, jnp.int32)

    n_gt = c_hi
    n_eq = c_lo - c_hi
    rem = CAP - n_gt

    def count_eq_le(icut_s):
        icut_v = jnp.full((LANES,), icut_s, jnp.int32)

        def body(i, acc):
            base = i * (LANES * UNROLL)
            for u in range(UNROLL):
                off = base + u * LANES
                v = col_v[pl.ds(off, LANES)]
                m = (v == thr_v) & ((lane_iota + off) <= icut_v)
                acc = acc + jnp.where(m, 1, 0).astype(jnp.int32)
            return acc

        return _lane_total(
            lax.fori_loop(0, N_CHUNK // UNROLL, body, zero_v)
        )

    def istep(_, lohi):
        lo, hi = lohi
        mid = lo + lax.shift_right_arithmetic(hi - lo + 1, 1)
        ok = count_eq_le(mid) >= rem
        return (jnp.where(ok, lo, mid), jnp.where(ok, mid, hi))

    def _tie_search():
        _, icut = lax.fori_loop(
            0, 13, istep, (jnp.int32(-1), jnp.int32(S_SEQ - 1))
        )
        return icut

    icut = lax.cond(n_eq == rem, lambda: jnp.int32(S_SEQ - 1), _tie_search)
    icut_v = jnp.full((LANES,), icut, jnp.int32)

    one_f = jnp.ones((LANES,), jnp.float32)
    zero_f = jnp.zeros((LANES,), jnp.float32)

    def selbody(i, carry):
        base = i * (LANES * UNROLL)
        for u in range(UNROLL):
            off = base + u * LANES
            v = col_v[pl.ds(off, LANES)]
            take = (v > thr_v) | ((v == thr_v) & ((lane_iota + off) <= icut_v))
            sel_v[pl.ds(off, LANES)] = jnp.where(take, one_f, zero_f)
        return carry

    lax.fori_loop(0, N_CHUNK // UNROLL, selbody, jnp.int32(0))

    pltpu.sync_copy(sel_v, un_v.at[e])
    plsc.subcore_barrier()

    base = e * COLS_PER_SUB
    pltpu.sync_copy(un_v.at[:, pl.ds(base, COLS_PER_SUB)], stage_v)
    for k in range(COLS_PER_SUB // LANES):
        acc = stage_v[0, pl.ds(k * LANES, LANES)]
        for r in range(1, 16):
            acc = jnp.maximum(acc, stage_v[r, pl.ds(k * LANES, LANES)])
        sel_v[pl.ds(k * LANES, LANES)] = acc
    pltpu.sync_copy(sel_v.at[pl.ds(0, COLS_PER_SUB)], out_hbm.at[b, pl.ds(base, COLS_PER_SUB)])


_sc_mask = functools.partial(
    pl.kernel,
    out_type=jax.ShapeDtypeStruct((N_BATCH, S_SEQ), jnp.float32),
    mesh=plsc.VectorSubcoreMesh(core_axis_name="c", subcore_axis_name="s"),
    scratch_types=[
        pltpu.VMEM((S_SEQ,), jnp.int32),
        pltpu.VMEM((S_SEQ,), jnp.float32),
        pltpu.VMEM((16, COLS_PER_SUB), jnp.float32),
        pltpu.VMEM((LANES,), jnp.int32),
        pltpu.VMEM_SHARED((16, S_SEQ), jnp.float32),
    ],
)(_sc_mask_body)


@jax.jit
def kernel(x, W):
    xr = x.reshape(N_BATCH * S_SEQ, D_EMBED)
    wt = W.T

    n_tiles = (N_BATCH * S_SEQ) // ROW_TILE
    logits_r, probs_r, probsT_r = pl.pallas_call(
        _router_body,
        grid=(n_tiles,),
        in_specs=[
            pl.BlockSpec((ROW_TILE, D_EMBED), lambda i: (i, 0)),
            pl.BlockSpec((D_EMBED, N_EXP), lambda i: (0, 0)),
        ],
        out_specs=[
            pl.BlockSpec((ROW_TILE, N_EXP), lambda i: (i, 0)),
            pl.BlockSpec((ROW_TILE, N_EXP), lambda i: (i, 0)),
            pl.BlockSpec(
                (N_EXP, ROW_TILE),
                lambda i: (i // N_TILES_PER_B, i % N_TILES_PER_B),
            ),
        ],
        out_shape=[
            jax.ShapeDtypeStruct((N_BATCH * S_SEQ, N_EXP), jnp.float32),
            jax.ShapeDtypeStruct((N_BATCH * S_SEQ, N_EXP), jnp.float32),
            jax.ShapeDtypeStruct((N_BATCH * N_EXP, S_SEQ), jnp.int32),
        ],
    )(xr, wt)

    logits = logits_r.reshape(N_BATCH, S_SEQ, N_EXP)
    probs = probs_r.reshape(N_BATCH, S_SEQ, N_EXP)
    probsT = probsT_r.reshape(N_BATCH, N_EXP, S_SEQ)

    sel = _sc_mask(probsT)
    mask = jnp.concatenate(
        [sel[:, :, None], jnp.zeros((N_BATCH, S_SEQ, N_EXP - 1), jnp.float32)],
        axis=-1,
    )

    return (mask, probs, logits)


# speedup vs baseline: 1.0044x; 1.0018x over previous
import functools
import jax
import jax.numpy as jnp
from jax import lax
from jax.experimental import pallas as pl
from jax.experimental.pallas import tpu as pltpu
from jax.experimental.pallas import tpu_sc as plsc

D_EMBED = 2048
N_EXP = 16
CAP = 512
N_BATCH = 2
S_SEQ = 4096

ROW_TILE = 512
N_TILES_PER_B = S_SEQ // ROW_TILE

LANES = 16
N_CHUNK = S_SEQ // LANES
UNROLL = 8
COLS_PER_SUB = S_SEQ // 16


def _router_body(x_ref, wt_ref, logits_ref, probs_ref, probsT_ref):
    l = jnp.dot(x_ref[...], wt_ref[...], preferred_element_type=jnp.float32)
    m = jnp.max(l, axis=-1, keepdims=True)
    e = jnp.exp(l - m)
    p = e / jnp.sum(e, axis=-1, keepdims=True)
    logits_ref[...] = l
    probs_ref[...] = p
    probsT_ref[...] = jax.lax.bitcast_convert_type(p.T, jnp.int32)


def _sc_mask_body(probsT_hbm, out_hbm, col_v, sel_v, stage_v, un_v):
    b = lax.axis_index("c")
    e = lax.axis_index("s")

    pltpu.sync_copy(probsT_hbm.at[b, e], col_v)

    zero_v = jnp.zeros((LANES,), jnp.int32)
    lane_iota = lax.iota(jnp.int32, LANES)

    def _lane_total(acc):
        tot = acc[0]
        for j in range(1, LANES):
            tot = tot + acc[j]
        return tot

    def count_ge(thr_s):
        thr_v = jnp.full((LANES,), thr_s, jnp.int32)

        def body(i, acc):
            base = i * (LANES * UNROLL)
            for u in range(UNROLL):
                v = col_v[pl.ds(base + u * LANES, LANES)]
                acc = acc + jnp.where(v >= thr_v, 1, 0).astype(jnp.int32)
            return acc

        return _lane_total(
            lax.fori_loop(0, N_CHUNK // UNROLL, body, zero_v)
        )

    def bstep(_, state):
        lo, hi, c_lo, c_hi = state
        mid = lo + lax.shift_right_arithmetic(hi - lo, 1)
        c = count_ge(mid)
        ge = c >= CAP
        return (
            jnp.where(ge, mid, lo),
            jnp.where(ge, hi, mid),
            jnp.where(ge, c, c_lo),
            jnp.where(ge, c_hi, c),
        )

    lo, _, c_lo, c_hi = lax.fori_loop(
        0,
        30,
        bstep,
        (jnp.int32(0), jnp.int32(0x3F800001), jnp.int32(S_SEQ), jnp.int32(0)),
    )
    thr_v = jnp.full((LANES,), lo, jnp.int32)

    n_gt = c_hi
    n_eq = c_lo - c_hi
    rem = CAP - n_gt

    def count_eq_le(icut_s):
        icut_v = jnp.full((LANES,), icut_s, jnp.int32)

        def body(i, acc):
            base = i * (LANES * UNROLL)
            for u in range(UNROLL):
                off = base + u * LANES
                v = col_v[pl.ds(off, LANES)]
                m = (v == thr_v) & ((lane_iota + off) <= icut_v)
                acc = acc + jnp.where(m, 1, 0).astype(jnp.int32)
            return acc

        return _lane_total(
            lax.fori_loop(0, N_CHUNK // UNROLL, body, zero_v)
        )

    def istep(_, lohi):
        lo, hi = lohi
        mid = lo + lax.shift_right_arithmetic(hi - lo + 1, 1)
        ok = count_eq_le(mid) >= rem
        return (jnp.where(ok, lo, mid), jnp.where(ok, mid, hi))

    def _tie_search():
        _, icut = lax.fori_loop(
            0, 13, istep, (jnp.int32(-1), jnp.int32(S_SEQ - 1))
        )
        return icut

    icut = lax.cond(n_eq == rem, lambda: jnp.int32(S_SEQ - 1), _tie_search)
    icut_v = jnp.full((LANES,), icut, jnp.int32)

    one_f = jnp.ones((LANES,), jnp.float32)
    zero_f = jnp.zeros((LANES,), jnp.float32)

    def selbody(i, carry):
        base = i * (LANES * UNROLL)
        for u in range(UNROLL):
            off = base + u * LANES
            v = col_v[pl.ds(off, LANES)]
            take = (v > thr_v) | ((v == thr_v) & ((lane_iota + off) <= icut_v))
            sel_v[pl.ds(off, LANES)] = jnp.where(take, one_f, zero_f)
        return carry

    lax.fori_loop(0, N_CHUNK // UNROLL, selbody, jnp.int32(0))

    pltpu.sync_copy(sel_v, un_v.at[e])
    plsc.subcore_barrier()

    base = e * COLS_PER_SUB
    pltpu.sync_copy(un_v.at[:, pl.ds(base, COLS_PER_SUB)], stage_v)
    for k in range(COLS_PER_SUB // LANES):
        acc = stage_v[0, pl.ds(k * LANES, LANES)]
        for r in range(1, 16):
            acc = jnp.maximum(acc, stage_v[r, pl.ds(k * LANES, LANES)])
        sel_v[pl.ds(k * LANES, LANES)] = acc
    pltpu.sync_copy(sel_v.at[pl.ds(0, COLS_PER_SUB)], out_hbm.at[b, pl.ds(base, COLS_PER_SUB)])


_sc_mask = functools.partial(
    pl.kernel,
    out_type=jax.ShapeDtypeStruct((N_BATCH, S_SEQ), jnp.float32),
    mesh=plsc.VectorSubcoreMesh(core_axis_name="c", subcore_axis_name="s"),
    scratch_types=[
        pltpu.VMEM((S_SEQ,), jnp.int32),
        pltpu.VMEM((S_SEQ,), jnp.float32),
        pltpu.VMEM((16, COLS_PER_SUB), jnp.float32),
        pltpu.VMEM_SHARED((16, S_SEQ), jnp.float32),
    ],
)(_sc_mask_body)


@jax.jit
def kernel(x, W):
    xr = x.reshape(N_BATCH * S_SEQ, D_EMBED)
    wt = W.T

    n_tiles = (N_BATCH * S_SEQ) // ROW_TILE
    logits_r, probs_r, probsT_r = pl.pallas_call(
        _router_body,
        grid=(n_tiles,),
        in_specs=[
            pl.BlockSpec((ROW_TILE, D_EMBED), lambda i: (i, 0)),
            pl.BlockSpec((D_EMBED, N_EXP), lambda i: (0, 0)),
        ],
        out_specs=[
            pl.BlockSpec((ROW_TILE, N_EXP), lambda i: (i, 0)),
            pl.BlockSpec((ROW_TILE, N_EXP), lambda i: (i, 0)),
            pl.BlockSpec(
                (N_EXP, ROW_TILE),
                lambda i: (i // N_TILES_PER_B, i % N_TILES_PER_B),
            ),
        ],
        out_shape=[
            jax.ShapeDtypeStruct((N_BATCH * S_SEQ, N_EXP), jnp.float32),
            jax.ShapeDtypeStruct((N_BATCH * S_SEQ, N_EXP), jnp.float32),
            jax.ShapeDtypeStruct((N_BATCH * N_EXP, S_SEQ), jnp.int32),
        ],
    )(xr, wt)

    logits = logits_r.reshape(N_BATCH, S_SEQ, N_EXP)
    probs = probs_r.reshape(N_BATCH, S_SEQ, N_EXP)
    probsT = probsT_r.reshape(N_BATCH, N_EXP, S_SEQ)

    sel = _sc_mask(probsT)
    mask = jnp.concatenate(
        [sel[:, :, None], jnp.zeros((N_BATCH, S_SEQ, N_EXP - 1), jnp.float32)],
        axis=-1,
    )

    return (mask, probs, logits)
